# baseline (speedup 1.0000x reference)
import functools

import jax
import jax.numpy as jnp
from jax import lax
from jax.experimental import pallas as pl
from jax.experimental.pallas import tpu as pltpu
from jax.experimental.pallas import tpu_sc as plsc

BATCH = 4096
HIST = 200
EMBED = 128
TOT = BATCH * HIST
NC, NS = 2, 16
NW = NC * NS
ROWS_PW = TOT // NW
G = 128
CH = ROWS_PW // G
NBUF = 4
SBUF = 2


def _body(x_hbm, table_hbm, out_hbm, idx_v, shared_v, bufs,
          gsems, csems, dsems, wsem):
    wid = lax.axis_index("s") * NC + lax.axis_index("c")
    sid = lax.axis_index("s")
    idx_row0 = wid * CH
    out_row0 = wid * ROWS_PW

    pltpu.sync_copy(x_hbm.at[pl.ds(idx_row0, CH)], idx_v)

    for b in range(NBUF):
        pltpu.async_copy(table_hbm.at[idx_v.at[b]], bufs[b], gsems[b])

    def slot_rows(k):
        return pl.multiple_of((sid * SBUF + k) * G, G)

    @pl.loop(0, CH // NBUF)
    def _step(s):
        for b in range(NBUF):
            j = s * NBUF + b
            off = pl.multiple_of(out_row0 + j * G, G)
            pltpu.make_async_copy(
                table_hbm.at[idx_v.at[j]], bufs[b], gsems[b]).wait()
            if b == 0:
                pltpu.async_copy(bufs[b], out_hbm.at[pl.ds(off, G)],
                                 wsem).wait()
            else:
                k = (b - 1) % SBUF
                srow = slot_rows(k)
                def _wait_drain():
                    pltpu.make_async_copy(
                        shared_v.at[pl.ds(srow, G)],
                        out_hbm.at[pl.ds(off, G)], dsems[k]).wait()

                if b - 1 < SBUF:
                    pl.when(s > 0)(_wait_drain)
                else:
                    _wait_drain()
                pltpu.async_copy(bufs[b], shared_v.at[pl.ds(srow, G)],
                                 csems[k]).wait()
                pltpu.async_copy(shared_v.at[pl.ds(srow, G)],
                                 out_hbm.at[pl.ds(off, G)], dsems[k])
            nxt = j + NBUF

            @pl.when(nxt < CH)
            def _():
                pltpu.async_copy(
                    table_hbm.at[idx_v.at[nxt]], bufs[b], gsems[b])

    for k in range(SBUF):
        pltpu.make_async_copy(
            shared_v.at[pl.ds(slot_rows(k), G)],
            out_hbm.at[pl.ds(pl.multiple_of(out_row0, G), G)],
            dsems[k]).wait()


def _flat_body(x_hbm, table_hbm, out_hbm, idx_v, shared_v, *rest):
    bufs = rest[:NBUF]
    gsems = rest[NBUF:2 * NBUF]
    csems = rest[2 * NBUF:2 * NBUF + SBUF]
    dsems = rest[2 * NBUF + SBUF:2 * NBUF + 2 * SBUF]
    wsem = rest[2 * NBUF + 2 * SBUF]
    _body(x_hbm, table_hbm, out_hbm, idx_v, shared_v, bufs,
          gsems, csems, dsems, wsem)


@jax.jit
def _embed(x2d, table):
    mesh = plsc.VectorSubcoreMesh(
        core_axis_name="c", subcore_axis_name="s",
        num_cores=NC, num_subcores=NS)
    scratch = (
        [pltpu.VMEM((CH, G), jnp.int32)]
        + [pltpu.VMEM_SHARED((NS * SBUF * G, EMBED), jnp.float32)]
        + [pltpu.VMEM((G, EMBED), jnp.float32) for _ in range(NBUF)]
        + [pltpu.SemaphoreType.DMA for _ in range(2 * NBUF + 2 * SBUF + 1)]
    )
    run = pl.kernel(
        _flat_body,
        out_type=jax.ShapeDtypeStruct((TOT, EMBED), jnp.float32),
        mesh=mesh,
        scratch_types=scratch,
    )
    return run(x2d, table)


def kernel(x, table):
    x2d = x.reshape(TOT // G, G).astype(jnp.int32)
    out = _embed(x2d, table)
    return out.reshape(BATCH, HIST, EMBED)


# speedup vs baseline: 1.0004x; 1.0004x over previous
import functools

import jax
import jax.numpy as jnp
from jax import lax
from jax.experimental import pallas as pl
from jax.experimental.pallas import tpu as pltpu
from jax.experimental.pallas import tpu_sc as plsc

BATCH = 4096
HIST = 200
EMBED = 128
TOT = BATCH * HIST
NC, NS = 2, 16
NW = NC * NS
ROWS_PW = TOT // NW
G = 128
CH = ROWS_PW // G
NBUF = 4
DIRECT_BS = (0, 2)
SPMEM_BS = tuple(b for b in range(NBUF) if b not in DIRECT_BS)
SBUF = 2


def _body(x_hbm, table_hbm, out_hbm, idx_v, shared_v, bufs,
          gsems, csems, dsems, wsem):
    wid = lax.axis_index("s") * NC + lax.axis_index("c")
    sid = lax.axis_index("s")
    idx_row0 = wid * CH
    out_row0 = wid * ROWS_PW

    pltpu.sync_copy(x_hbm.at[pl.ds(idx_row0, CH)], idx_v)

    for b in range(NBUF):
        pltpu.async_copy(table_hbm.at[idx_v.at[b]], bufs[b], gsems[b])

    def slot_rows(k):
        return pl.multiple_of((sid * SBUF + k) * G, G)

    @pl.loop(0, CH // NBUF)
    def _step(s):
        for b in range(NBUF):
            j = s * NBUF + b
            off = pl.multiple_of(out_row0 + j * G, G)
            pltpu.make_async_copy(
                table_hbm.at[idx_v.at[j]], bufs[b], gsems[b]).wait()
            if b in DIRECT_BS:
                pltpu.async_copy(bufs[b], out_hbm.at[pl.ds(off, G)],
                                 wsem).wait()
            else:
                i = SPMEM_BS.index(b)
                k = i % SBUF
                srow = slot_rows(k)
                def _wait_drain():
                    pltpu.make_async_copy(
                        shared_v.at[pl.ds(srow, G)],
                        out_hbm.at[pl.ds(off, G)], dsems[k]).wait()

                if i < SBUF:
                    pl.when(s > 0)(_wait_drain)
                else:
                    _wait_drain()
                pltpu.async_copy(bufs[b], shared_v.at[pl.ds(srow, G)],
                                 csems[k]).wait()
                pltpu.async_copy(shared_v.at[pl.ds(srow, G)],
                                 out_hbm.at[pl.ds(off, G)], dsems[k])
            nxt = j + NBUF

            @pl.when(nxt < CH)
            def _():
                pltpu.async_copy(
                    table_hbm.at[idx_v.at[nxt]], bufs[b], gsems[b])

    for k in range(SBUF):
        pltpu.make_async_copy(
            shared_v.at[pl.ds(slot_rows(k), G)],
            out_hbm.at[pl.ds(pl.multiple_of(out_row0, G), G)],
            dsems[k]).wait()


def _flat_body(x_hbm, table_hbm, out_hbm, idx_v, shared_v, *rest):
    bufs = rest[:NBUF]
    gsems = rest[NBUF:2 * NBUF]
    csems = rest[2 * NBUF:2 * NBUF + SBUF]
    dsems = rest[2 * NBUF + SBUF:2 * NBUF + 2 * SBUF]
    wsem = rest[2 * NBUF + 2 * SBUF]
    _body(x_hbm, table_hbm, out_hbm, idx_v, shared_v, bufs,
          gsems, csems, dsems, wsem)


@jax.jit
def _embed(x2d, table):
    mesh = plsc.VectorSubcoreMesh(
        core_axis_name="c", subcore_axis_name="s",
        num_cores=NC, num_subcores=NS)
    scratch = (
        [pltpu.VMEM((CH, G), jnp.int32)]
        + [pltpu.VMEM_SHARED((NS * SBUF * G, EMBED), jnp.float32)]
        + [pltpu.VMEM((G, EMBED), jnp.float32) for _ in range(NBUF)]
        + [pltpu.SemaphoreType.DMA for _ in range(2 * NBUF + 2 * SBUF + 1)]
    )
    run = pl.kernel(
        _flat_body,
        out_type=jax.ShapeDtypeStruct((TOT, EMBED), jnp.float32),
        mesh=mesh,
        scratch_types=scratch,
    )
    return run(x2d, table)


def kernel(x, table):
    x2d = x.reshape(TOT // G, G).astype(jnp.int32)
    out = _embed(x2d, table)
    return out.reshape(BATCH, HIST, EMBED)
